# baseline (speedup 1.0000x reference)
import functools

import jax
import jax.numpy as jnp
from jax import lax
from jax.experimental import pallas as pl
from jax.experimental.pallas import tpu as pltpu
from jax.experimental.pallas import tpu_sc as plsc

N = 10000
E = 160000
D = 256
DE = 16
H = 256

NODE_BLK = 1000
EDGE_BLK = 1600

NC = 2
NS = 16
NW = NC * NS
GB = 200
SB = 200
HH = H // 2
NSTRIPE = N // NS


def _node_pre_body(nf_ref, ws_ref, bs_ref, wd_ref, bd_ref, s_ref, d_ref):
    nf = nf_ref[...]
    s = jnp.dot(nf, ws_ref[...], preferred_element_type=jnp.float32) + bs_ref[...]
    d = jnp.dot(nf, wd_ref[...], preferred_element_type=jnp.float32) + bd_ref[...]
    s_ref[...] = s.astype(jnp.bfloat16)
    d_ref[...] = d.astype(jnp.bfloat16)


def _node_precompute(node_feat, Ws, bs, Wd, bd):
    return pl.pallas_call(
        _node_pre_body,
        grid=(N // NODE_BLK,),
        in_specs=[
            pl.BlockSpec((NODE_BLK, D), lambda i: (i, 0)),
            pl.BlockSpec((D, H), lambda i: (0, 0)),
            pl.BlockSpec((H,), lambda i: (0,)),
            pl.BlockSpec((D, H), lambda i: (0, 0)),
            pl.BlockSpec((H,), lambda i: (0,)),
        ],
        out_specs=[
            pl.BlockSpec((NODE_BLK, H), lambda i: (i, 0)),
            pl.BlockSpec((NODE_BLK, H), lambda i: (i, 0)),
        ],
        out_shape=[
            jax.ShapeDtypeStruct((N, H), jnp.bfloat16),
            jax.ShapeDtypeStruct((N, H), jnp.bfloat16),
        ],
    )(node_feat, Ws, bs, Wd, bd)


def _sc_gather(s, d, src, dst):
    W32 = H // 2
    per_w = E // NW
    n_it = per_w // GB

    mesh = plsc.VectorSubcoreMesh(core_axis_name="c", subcore_axis_name="s")

    @functools.partial(
        pl.kernel,
        mesh=mesh,
        out_type=[
            jax.ShapeDtypeStruct((E, W32), jnp.float32),
            jax.ShapeDtypeStruct((E, W32), jnp.float32),
        ],
        scratch_types=[
            pltpu.VMEM((GB,), jnp.int32),
            pltpu.VMEM((GB,), jnp.int32),
            pltpu.VMEM((GB, W32), jnp.float32),
            pltpu.VMEM((GB, W32), jnp.float32),
            pltpu.SemaphoreType.DMA,
            pltpu.SemaphoreType.DMA,
        ],
    )
    def k(s_hbm, d_hbm, src_hbm, dst_hbm, gs_hbm, gd_hbm,
          idx_s, idx_d, rows_s, rows_d, sem_s, sem_d):
        wid = lax.axis_index("s") * NC + lax.axis_index("c")
        base = wid * per_w

        def body(i, carry):
            off = base + i * GB
            pltpu.sync_copy(src_hbm.at[pl.ds(off, GB)], idx_s)
            pltpu.sync_copy(dst_hbm.at[pl.ds(off, GB)], idx_d)
            cp_s = pltpu.async_copy(s_hbm.at[idx_s], rows_s, sem_s)
            cp_d = pltpu.async_copy(d_hbm.at[idx_d], rows_d, sem_d)
            cp_s.wait()
            cp_d.wait()
            pltpu.sync_copy(rows_s, gs_hbm.at[pl.ds(off, GB)])
            pltpu.sync_copy(rows_d, gd_hbm.at[pl.ds(off, GB)])
            return carry

        lax.fori_loop(0, n_it, body, 0)

    return k(s, d, src, dst)


def _edge_mlp_body(ea_ref, gs_ref, gd_ref, we1_ref, be1_ref, we2_ref, be2_ref,
                   wt1_ref, bt1_ref, wt2_ref, bt2_ref, m_ref):
    h1 = jnp.maximum(jnp.dot(ea_ref[...], we1_ref[...],
                             preferred_element_type=jnp.float32) + be1_ref[...], 0.0)
    ec = jnp.dot(h1.astype(jnp.bfloat16), we2_ref[...],
                 preferred_element_type=jnp.float32) + be2_ref[...]
    m1 = jnp.maximum(ec + gs_ref[...].astype(jnp.float32)
                     + gd_ref[...].astype(jnp.float32), 0.0)
    m2 = jnp.maximum(jnp.dot(m1.astype(jnp.bfloat16), wt1_ref[...],
                             preferred_element_type=jnp.float32) + bt1_ref[...], 0.0)
    m_ref[...] = jnp.dot(m2.astype(jnp.bfloat16), wt2_ref[...],
                         preferred_element_type=jnp.float32) + bt2_ref[...]


def _edge_mlp(edge_attr, gs, gd, We1, be1, We2, be2, Wt1, bt1, Wt2, bt2):
    return pl.pallas_call(
        _edge_mlp_body,
        grid=(E // EDGE_BLK,),
        in_specs=[
            pl.BlockSpec((EDGE_BLK, DE), lambda i: (i, 0)),
            pl.BlockSpec((EDGE_BLK, H), lambda i: (i, 0)),
            pl.BlockSpec((EDGE_BLK, H), lambda i: (i, 0)),

            pl.BlockSpec((DE, H), lambda i: (0, 0)),
            pl.BlockSpec((H,), lambda i: (0,)),
            pl.BlockSpec((H, H), lambda i: (0, 0)),
            pl.BlockSpec((H,), lambda i: (0,)),
            pl.BlockSpec((H, H), lambda i: (0, 0)),
            pl.BlockSpec((H,), lambda i: (0,)),
            pl.BlockSpec((H, D), lambda i: (0, 0)),
            pl.BlockSpec((D,), lambda i: (0,)),
        ],
        out_specs=pl.BlockSpec((EDGE_BLK, D), lambda i: (i, 0)),
        out_shape=jax.ShapeDtypeStruct((E, D), jnp.float32),
    )(edge_attr, gs, gd, We1, be1, We2, be2, Wt1, bt1, Wt2, bt2)


def _sc_scatter_add(m3, dst, zeros_stripe):
    per_t = E // NS
    n_it = per_t // SB

    mesh = plsc.VectorSubcoreMesh(core_axis_name="c", subcore_axis_name="s")

    @functools.partial(
        pl.kernel,
        mesh=mesh,
        out_type=jax.ShapeDtypeStruct((N, NC, HH), jnp.float32),
        scratch_types=[
            pltpu.VMEM((SB,), jnp.int32),
            pltpu.VMEM((SB, HH), jnp.float32),
            pltpu.VMEM_SHARED((N, HH), jnp.float32),
        ],
    )
    def k(m_hbm, dst_hbm, z_hbm, out_hbm, idx_v, blk_v, acc):
        c = lax.axis_index("c")
        t = lax.axis_index("s")

        pltpu.sync_copy(z_hbm, acc.at[pl.ds(t * NSTRIPE, NSTRIPE)])
        plsc.subcore_barrier()

        def body(i, carry):
            off = t * per_t + i * SB
            pltpu.sync_copy(dst_hbm.at[pl.ds(off, SB)], idx_v)
            pltpu.sync_copy(m_hbm.at[pl.ds(off, SB), c], blk_v)
            pltpu.sync_copy(blk_v, acc.at[idx_v], add=True)
            return carry

        lax.fori_loop(0, n_it, body, 0)
        plsc.subcore_barrier()

        pltpu.sync_copy(acc.at[pl.ds(t * NSTRIPE, NSTRIPE)],
                        out_hbm.at[pl.ds(t * NSTRIPE, NSTRIPE), c])

    return k(m3, dst, zeros_stripe)


def _final_body(nf_ref, agg_ref, wpd_ref, bpd_ref, wpe_ref, bpe_ref,
                wp_ref, bp_ref, out_ref):
    z = (jnp.dot(nf_ref[...], wpd_ref[...], preferred_element_type=jnp.float32)
         + bpd_ref[...]
         + jnp.dot(agg_ref[...], wpe_ref[...], preferred_element_type=jnp.float32)
         + bpe_ref[...])
    out_ref[...] = jnp.dot(jnp.maximum(z, 0.0), wp_ref[...],
                           preferred_element_type=jnp.float32) + bp_ref[...]


def _final_stage(node_feat, agg, Wpd, bpd, Wpe, bpe, Wp, bp):
    return pl.pallas_call(
        _final_body,
        grid=(N // NODE_BLK,),
        in_specs=[
            pl.BlockSpec((NODE_BLK, D), lambda i: (i, 0)),
            pl.BlockSpec((NODE_BLK, D), lambda i: (i, 0)),
            pl.BlockSpec((D, H), lambda i: (0, 0)),
            pl.BlockSpec((H,), lambda i: (0,)),
            pl.BlockSpec((D, H), lambda i: (0, 0)),
            pl.BlockSpec((H,), lambda i: (0,)),
            pl.BlockSpec((H, D), lambda i: (0, 0)),
            pl.BlockSpec((D,), lambda i: (0,)),
        ],
        out_specs=pl.BlockSpec((NODE_BLK, D), lambda i: (i, 0)),
        out_shape=jax.ShapeDtypeStruct((N, D), jnp.float32),
    )(node_feat, agg, Wpd, bpd, Wpe, bpe, Wp, bp)


def kernel(node_feat, edge_index, edge_attr, We1, be1, We2, be2, Ws, bs, Wd, bd,
           Wt1, bt1, Wt2, bt2, Wpd, bpd, Wpe, bpe, Wp, bp):
    src = edge_index[0]
    dst = edge_index[1]
    s, d = _node_precompute(node_feat, Ws, bs, Wd, bd)
    s32 = lax.bitcast_convert_type(s.reshape(N, H // 2, 2), jnp.float32)
    d32 = lax.bitcast_convert_type(d.reshape(N, H // 2, 2), jnp.float32)
    gs32, gd32 = _sc_gather(s32, d32, src, dst)
    gs = lax.bitcast_convert_type(gs32, jnp.bfloat16).reshape(E, H)
    gd = lax.bitcast_convert_type(gd32, jnp.bfloat16).reshape(E, H)
    m = _edge_mlp(edge_attr, gs, gd, We1, be1,
                  We2.astype(jnp.bfloat16), be2, Wt1.astype(jnp.bfloat16), bt1,
                  Wt2.astype(jnp.bfloat16), bt2)
    zeros_stripe = jnp.zeros((NSTRIPE, HH), jnp.float32)
    agg3 = _sc_scatter_add(m.reshape(E, NC, HH), dst, zeros_stripe)
    agg = agg3.reshape(N, D)
    return _final_stage(node_feat, agg, Wpd, bpd, Wpe, bpe, Wp, bp)


# speedup vs baseline: 2.4556x; 2.4556x over previous
import functools

import jax
import jax.numpy as jnp
from jax import lax
from jax.experimental import pallas as pl
from jax.experimental.pallas import tpu as pltpu
from jax.experimental.pallas import tpu_sc as plsc

N = 10000
E = 160000
D = 256
DE = 16
H = 256

NODE_BLK = 1000
EDGE_BLK = 1600

NC = 2
NS = 16
NW = NC * NS
GB = 200
SB = 200
HH = H // 2
NSTRIPE = N // NS


def _node_pre_body(nf_ref, ws_ref, bs_ref, wd_ref, bd_ref, s_ref, d_ref):
    nf = nf_ref[...]
    s_ref[...] = jnp.dot(nf, ws_ref[...], preferred_element_type=jnp.float32) + bs_ref[...]
    d_ref[...] = jnp.dot(nf, wd_ref[...], preferred_element_type=jnp.float32) + bd_ref[...]


def _node_precompute(node_feat, Ws, bs, Wd, bd):
    return pl.pallas_call(
        _node_pre_body,
        grid=(N // NODE_BLK,),
        in_specs=[
            pl.BlockSpec((NODE_BLK, D), lambda i: (i, 0)),
            pl.BlockSpec((D, H), lambda i: (0, 0)),
            pl.BlockSpec((H,), lambda i: (0,)),
            pl.BlockSpec((D, H), lambda i: (0, 0)),
            pl.BlockSpec((H,), lambda i: (0,)),
        ],
        out_specs=[
            pl.BlockSpec((NODE_BLK, H), lambda i: (i, 0)),
            pl.BlockSpec((NODE_BLK, H), lambda i: (i, 0)),
        ],
        out_shape=[
            jax.ShapeDtypeStruct((N, H), jnp.float32),
            jax.ShapeDtypeStruct((N, H), jnp.float32),
        ],
    )(node_feat, Ws, bs, Wd, bd)


def _sc_gather(s, d, src, dst):
    per_w = E // NW
    n_it = per_w // GB

    mesh = plsc.VectorSubcoreMesh(core_axis_name="c", subcore_axis_name="s")

    @functools.partial(
        pl.kernel,
        mesh=mesh,
        out_type=[
            jax.ShapeDtypeStruct((E, H), jnp.float32),
            jax.ShapeDtypeStruct((E, H), jnp.float32),
        ],
        scratch_types=[
            pltpu.VMEM((GB,), jnp.int32),
            pltpu.VMEM((GB,), jnp.int32),
            pltpu.VMEM((GB, H), jnp.float32),
            pltpu.VMEM((GB, H), jnp.float32),
            pltpu.SemaphoreType.DMA,
            pltpu.SemaphoreType.DMA,
        ],
    )
    def k(s_hbm, d_hbm, src_hbm, dst_hbm, gs_hbm, gd_hbm,
          idx_s, idx_d, rows_s, rows_d, sem_s, sem_d):
        wid = lax.axis_index("s") * NC + lax.axis_index("c")
        base = wid * per_w

        def body(i, carry):
            off = base + i * GB
            pltpu.sync_copy(src_hbm.at[pl.ds(off, GB)], idx_s)
            pltpu.sync_copy(dst_hbm.at[pl.ds(off, GB)], idx_d)
            cp_s = pltpu.async_copy(s_hbm.at[idx_s], rows_s, sem_s)
            cp_d = pltpu.async_copy(d_hbm.at[idx_d], rows_d, sem_d)
            cp_s.wait()
            cp_d.wait()
            pltpu.sync_copy(rows_s, gs_hbm.at[pl.ds(off, GB)])
            pltpu.sync_copy(rows_d, gd_hbm.at[pl.ds(off, GB)])
            return carry

        lax.fori_loop(0, n_it, body, 0)

    return k(s, d, src, dst)


def _edge_mlp_body(ea_ref, gs_ref, gd_ref, we1_ref, be1_ref, we2_ref, be2_ref,
                   wt1_ref, bt1_ref, wt2_ref, bt2_ref, m_ref):
    h1 = jnp.maximum(jnp.dot(ea_ref[...], we1_ref[...],
                             preferred_element_type=jnp.float32) + be1_ref[...], 0.0)
    ec = jnp.dot(h1.astype(jnp.bfloat16), we2_ref[...],
                 preferred_element_type=jnp.float32) + be2_ref[...]
    m1 = jnp.maximum(ec + gs_ref[...] + gd_ref[...], 0.0)
    m2 = jnp.maximum(jnp.dot(m1.astype(jnp.bfloat16), wt1_ref[...],
                             preferred_element_type=jnp.float32) + bt1_ref[...], 0.0)
    m_ref[...] = jnp.dot(m2.astype(jnp.bfloat16), wt2_ref[...],
                         preferred_element_type=jnp.float32) + bt2_ref[...]


def _edge_mlp(edge_attr, gs, gd, We1, be1, We2, be2, Wt1, bt1, Wt2, bt2):
    return pl.pallas_call(
        _edge_mlp_body,
        grid=(E // EDGE_BLK,),
        in_specs=[
            pl.BlockSpec((EDGE_BLK, DE), lambda i: (i, 0)),
            pl.BlockSpec((EDGE_BLK, H), lambda i: (i, 0)),
            pl.BlockSpec((EDGE_BLK, H), lambda i: (i, 0)),

            pl.BlockSpec((DE, H), lambda i: (0, 0)),
            pl.BlockSpec((H,), lambda i: (0,)),
            pl.BlockSpec((H, H), lambda i: (0, 0)),
            pl.BlockSpec((H,), lambda i: (0,)),
            pl.BlockSpec((H, H), lambda i: (0, 0)),
            pl.BlockSpec((H,), lambda i: (0,)),
            pl.BlockSpec((H, D), lambda i: (0, 0)),
            pl.BlockSpec((D,), lambda i: (0,)),
        ],
        out_specs=pl.BlockSpec((EDGE_BLK, D), lambda i: (i, 0)),
        out_shape=jax.ShapeDtypeStruct((E, D), jnp.float32),
    )(edge_attr, gs, gd, We1, be1, We2, be2, Wt1, bt1, Wt2, bt2)


def _sc_scatter_add(m3, dst, zeros_stripe):
    per_t = E // NS
    n_it = per_t // SB

    mesh = plsc.VectorSubcoreMesh(core_axis_name="c", subcore_axis_name="s")

    @functools.partial(
        pl.kernel,
        mesh=mesh,
        out_type=jax.ShapeDtypeStruct((N, NC, HH), jnp.float32),
        scratch_types=[
            pltpu.VMEM((SB,), jnp.int32),
            pltpu.VMEM((SB, HH), jnp.float32),
            pltpu.VMEM_SHARED((N, HH), jnp.float32),
        ],
    )
    def k(m_hbm, dst_hbm, z_hbm, out_hbm, idx_v, blk_v, acc):
        c = lax.axis_index("c")
        t = lax.axis_index("s")

        pltpu.sync_copy(z_hbm, acc.at[pl.ds(t * NSTRIPE, NSTRIPE)])
        plsc.subcore_barrier()

        def body(i, carry):
            off = t * per_t + i * SB
            pltpu.sync_copy(dst_hbm.at[pl.ds(off, SB)], idx_v)
            pltpu.sync_copy(m_hbm.at[pl.ds(off, SB), c], blk_v)
            pltpu.sync_copy(blk_v, acc.at[idx_v], add=True)
            return carry

        lax.fori_loop(0, n_it, body, 0)
        plsc.subcore_barrier()

        pltpu.sync_copy(acc.at[pl.ds(t * NSTRIPE, NSTRIPE)],
                        out_hbm.at[pl.ds(t * NSTRIPE, NSTRIPE), c])

    return k(m3, dst, zeros_stripe)


def _final_body(nf_ref, agg_ref, wpd_ref, bpd_ref, wpe_ref, bpe_ref,
                wp_ref, bp_ref, out_ref):
    z = (jnp.dot(nf_ref[...], wpd_ref[...], preferred_element_type=jnp.float32)
         + bpd_ref[...]
         + jnp.dot(agg_ref[...], wpe_ref[...], preferred_element_type=jnp.float32)
         + bpe_ref[...])
    out_ref[...] = jnp.dot(jnp.maximum(z, 0.0), wp_ref[...],
                           preferred_element_type=jnp.float32) + bp_ref[...]


def _final_stage(node_feat, agg, Wpd, bpd, Wpe, bpe, Wp, bp):
    return pl.pallas_call(
        _final_body,
        grid=(N // NODE_BLK,),
        in_specs=[
            pl.BlockSpec((NODE_BLK, D), lambda i: (i, 0)),
            pl.BlockSpec((NODE_BLK, D), lambda i: (i, 0)),
            pl.BlockSpec((D, H), lambda i: (0, 0)),
            pl.BlockSpec((H,), lambda i: (0,)),
            pl.BlockSpec((D, H), lambda i: (0, 0)),
            pl.BlockSpec((H,), lambda i: (0,)),
            pl.BlockSpec((H, D), lambda i: (0, 0)),
            pl.BlockSpec((D,), lambda i: (0,)),
        ],
        out_specs=pl.BlockSpec((NODE_BLK, D), lambda i: (i, 0)),
        out_shape=jax.ShapeDtypeStruct((N, D), jnp.float32),
    )(node_feat, agg, Wpd, bpd, Wpe, bpe, Wp, bp)


def kernel(node_feat, edge_index, edge_attr, We1, be1, We2, be2, Ws, bs, Wd, bd,
           Wt1, bt1, Wt2, bt2, Wpd, bpd, Wpe, bpe, Wp, bp):
    src = edge_index[0]
    dst = edge_index[1]
    s, d = _node_precompute(node_feat, Ws, bs, Wd, bd)
    gs, gd = _sc_gather(s, d, src, dst)
    m = _edge_mlp(edge_attr, gs, gd, We1, be1,
                  We2.astype(jnp.bfloat16), be2, Wt1.astype(jnp.bfloat16), bt1,
                  Wt2.astype(jnp.bfloat16), bt2)
    zeros_stripe = jnp.zeros((NSTRIPE, HH), jnp.float32)
    agg3 = _sc_scatter_add(m.reshape(E, NC, HH), dst, zeros_stripe)
    agg = agg3.reshape(N, D)
    return _final_stage(node_feat, agg, Wpd, bpd, Wpe, bpe, Wp, bp)


# speedup vs baseline: 2.6306x; 1.0713x over previous
import functools

import jax
import jax.numpy as jnp
from jax import lax
from jax.experimental import pallas as pl
from jax.experimental.pallas import tpu as pltpu
from jax.experimental.pallas import tpu_sc as plsc

N = 10000
E = 160000
D = 256
DE = 16
H = 256

K = 5
EC = E // K

NODE_BLK = 1000
EDGE_BLK = 1600

NC = 2
NS = 16
NW = NC * NS
GB = 200
SB = 200
HH = H // 2
NSTRIPE = N // NS


def _node_pre_body(nf_ref, ws_ref, bs_ref, wd_ref, bd_ref, s_ref, d_ref):
    nf = nf_ref[...]
    s_ref[...] = jnp.dot(nf, ws_ref[...], preferred_element_type=jnp.float32) + bs_ref[...]
    d_ref[...] = jnp.dot(nf, wd_ref[...], preferred_element_type=jnp.float32) + bd_ref[...]


def _node_precompute(node_feat, Ws, bs, Wd, bd):
    return pl.pallas_call(
        _node_pre_body,
        grid=(N // NODE_BLK,),
        in_specs=[
            pl.BlockSpec((NODE_BLK, D), lambda i: (i, 0)),
            pl.BlockSpec((D, H), lambda i: (0, 0)),
            pl.BlockSpec((H,), lambda i: (0,)),
            pl.BlockSpec((D, H), lambda i: (0, 0)),
            pl.BlockSpec((H,), lambda i: (0,)),
        ],
        out_specs=[
            pl.BlockSpec((NODE_BLK, H), lambda i: (i, 0)),
            pl.BlockSpec((NODE_BLK, H), lambda i: (i, 0)),
        ],
        out_shape=[
            jax.ShapeDtypeStruct((N, H), jnp.float32),
            jax.ShapeDtypeStruct((N, H), jnp.float32),
        ],
    )(node_feat, Ws, bs, Wd, bd)


def _sc_gather(s, d, src_c, dst_c):
    per_w = EC // NW
    n_it = per_w // GB

    mesh = plsc.VectorSubcoreMesh(core_axis_name="c", subcore_axis_name="s")

    @functools.partial(
        pl.kernel,
        mesh=mesh,
        out_type=[
            jax.ShapeDtypeStruct((EC, H), jnp.float32),
            jax.ShapeDtypeStruct((EC, H), jnp.float32),
        ],
        scratch_types=[
            pltpu.VMEM((GB,), jnp.int32),
            pltpu.VMEM((GB,), jnp.int32),
            pltpu.VMEM((GB, H), jnp.float32),
            pltpu.VMEM((GB, H), jnp.float32),
            pltpu.SemaphoreType.DMA,
            pltpu.SemaphoreType.DMA,
        ],
    )
    def k(s_hbm, d_hbm, src_hbm, dst_hbm, gs_hbm, gd_hbm,
          idx_s, idx_d, rows_s, rows_d, sem_s, sem_d):
        wid = lax.axis_index("s") * NC + lax.axis_index("c")
        base = wid * per_w

        def body(i, carry):
            off = base + i * GB
            pltpu.sync_copy(src_hbm.at[pl.ds(off, GB)], idx_s)
            pltpu.sync_copy(dst_hbm.at[pl.ds(off, GB)], idx_d)
            cp_s = pltpu.async_copy(s_hbm.at[idx_s], rows_s, sem_s)
            cp_d = pltpu.async_copy(d_hbm.at[idx_d], rows_d, sem_d)
            cp_s.wait()
            cp_d.wait()
            pltpu.sync_copy(rows_s, gs_hbm.at[pl.ds(off, GB)])
            pltpu.sync_copy(rows_d, gd_hbm.at[pl.ds(off, GB)])
            return carry

        lax.fori_loop(0, n_it, body, 0)

    return k(s, d, src_c, dst_c)


def _edge_mlp_body(ea_ref, gs_ref, gd_ref, we1_ref, be1_ref, we2_ref, be2_ref,
                   wt1_ref, bt1_ref, wt2_ref, bt2_ref, m_ref):
    h1 = jnp.maximum(jnp.dot(ea_ref[...], we1_ref[...],
                             preferred_element_type=jnp.float32) + be1_ref[...], 0.0)
    ec = jnp.dot(h1.astype(jnp.bfloat16), we2_ref[...],
                 preferred_element_type=jnp.float32) + be2_ref[...]
    m1 = jnp.maximum(ec + gs_ref[...] + gd_ref[...], 0.0)
    m2 = jnp.maximum(jnp.dot(m1.astype(jnp.bfloat16), wt1_ref[...],
                             preferred_element_type=jnp.float32) + bt1_ref[...], 0.0)
    m_ref[...] = jnp.dot(m2.astype(jnp.bfloat16), wt2_ref[...],
                         preferred_element_type=jnp.float32) + bt2_ref[...]


def _edge_mlp(edge_attr_c, gs, gd, We1, be1, We2b, be2, Wt1b, bt1, Wt2b, bt2):
    return pl.pallas_call(
        _edge_mlp_body,
        grid=(EC // EDGE_BLK,),
        in_specs=[
            pl.BlockSpec((EDGE_BLK, DE), lambda i: (i, 0)),
            pl.BlockSpec((EDGE_BLK, H), lambda i: (i, 0)),
            pl.BlockSpec((EDGE_BLK, H), lambda i: (i, 0)),
            pl.BlockSpec((DE, H), lambda i: (0, 0)),
            pl.BlockSpec((H,), lambda i: (0,)),
            pl.BlockSpec((H, H), lambda i: (0, 0)),
            pl.BlockSpec((H,), lambda i: (0,)),
            pl.BlockSpec((H, H), lambda i: (0, 0)),
            pl.BlockSpec((H,), lambda i: (0,)),
            pl.BlockSpec((H, D), lambda i: (0, 0)),
            pl.BlockSpec((D,), lambda i: (0,)),
        ],
        out_specs=pl.BlockSpec((EDGE_BLK, D), lambda i: (i, 0)),
        out_shape=jax.ShapeDtypeStruct((EC, D), jnp.float32),
    )(edge_attr_c, gs, gd, We1, be1, We2b, be2, Wt1b, bt1, Wt2b, bt2)


def _sc_scatter_add(m3, dst_c, zeros_stripe):
    per_t = EC // NS
    n_it = per_t // SB

    mesh = plsc.VectorSubcoreMesh(core_axis_name="c", subcore_axis_name="s")

    @functools.partial(
        pl.kernel,
        mesh=mesh,
        out_type=jax.ShapeDtypeStruct((N, NC, HH), jnp.float32),
        scratch_types=[
            pltpu.VMEM((SB,), jnp.int32),
            pltpu.VMEM((SB, HH), jnp.float32),
            pltpu.VMEM_SHARED((N, HH), jnp.float32),
        ],
    )
    def k(m_hbm, dst_hbm, z_hbm, out_hbm, idx_v, blk_v, acc):
        c = lax.axis_index("c")
        t = lax.axis_index("s")

        pltpu.sync_copy(z_hbm, acc.at[pl.ds(t * NSTRIPE, NSTRIPE)])
        plsc.subcore_barrier()

        def body(i, carry):
            off = t * per_t + i * SB
            pltpu.sync_copy(dst_hbm.at[pl.ds(off, SB)], idx_v)
            pltpu.sync_copy(m_hbm.at[pl.ds(off, SB), c], blk_v)
            pltpu.sync_copy(blk_v, acc.at[idx_v], add=True)
            return carry

        lax.fori_loop(0, n_it, body, 0)
        plsc.subcore_barrier()

        pltpu.sync_copy(acc.at[pl.ds(t * NSTRIPE, NSTRIPE)],
                        out_hbm.at[pl.ds(t * NSTRIPE, NSTRIPE), c])

    return k(m3, dst_c, zeros_stripe)


def _final_body(nf_ref, *rest):
    agg_refs = rest[:K]
    wpd_ref, bpd_ref, wpe_ref, bpe_ref, wp_ref, bp_ref, out_ref = rest[K:]
    agg = agg_refs[0][...]
    for r in agg_refs[1:]:
        agg = agg + r[...]
    z = (jnp.dot(nf_ref[...], wpd_ref[...], preferred_element_type=jnp.float32)
         + bpd_ref[...]
         + jnp.dot(agg, wpe_ref[...], preferred_element_type=jnp.float32)
         + bpe_ref[...])
    out_ref[...] = jnp.dot(jnp.maximum(z, 0.0), wp_ref[...],
                           preferred_element_type=jnp.float32) + bp_ref[...]


def _final_stage(node_feat, aggs, Wpd, bpd, Wpe, bpe, Wp, bp):
    return pl.pallas_call(
        _final_body,
        grid=(N // NODE_BLK,),
        in_specs=[pl.BlockSpec((NODE_BLK, D), lambda i: (i, 0))]
        + [pl.BlockSpec((NODE_BLK, D), lambda i: (i, 0)) for _ in range(K)]
        + [
            pl.BlockSpec((D, H), lambda i: (0, 0)),
            pl.BlockSpec((H,), lambda i: (0,)),
            pl.BlockSpec((D, H), lambda i: (0, 0)),
            pl.BlockSpec((H,), lambda i: (0,)),
            pl.BlockSpec((H, D), lambda i: (0, 0)),
            pl.BlockSpec((D,), lambda i: (0,)),
        ],
        out_specs=pl.BlockSpec((NODE_BLK, D), lambda i: (i, 0)),
        out_shape=jax.ShapeDtypeStruct((N, D), jnp.float32),
    )(node_feat, *aggs, Wpd, bpd, Wpe, bpe, Wp, bp)


def kernel(node_feat, edge_index, edge_attr, We1, be1, We2, be2, Ws, bs, Wd, bd,
           Wt1, bt1, Wt2, bt2, Wpd, bpd, Wpe, bpe, Wp, bp):
    src = edge_index[0]
    dst = edge_index[1]
    s, d = _node_precompute(node_feat, Ws, bs, Wd, bd)
    We2b = We2.astype(jnp.bfloat16)
    Wt1b = Wt1.astype(jnp.bfloat16)
    Wt2b = Wt2.astype(jnp.bfloat16)
    zeros_stripe = jnp.zeros((NSTRIPE, HH), jnp.float32)

    aggs = []
    for kk in range(K):
        lo = kk * EC
        src_c = lax.slice(src, (lo,), (lo + EC,))
        dst_c = lax.slice(dst, (lo,), (lo + EC,))
        ea_c = lax.slice(edge_attr, (lo, 0), (lo + EC, DE))
        gs, gd = _sc_gather(s, d, src_c, dst_c)
        m = _edge_mlp(ea_c, gs, gd, We1, be1, We2b, be2, Wt1b, bt1, Wt2b, bt2)
        agg3 = _sc_scatter_add(m.reshape(EC, NC, HH), dst_c, zeros_stripe)
        aggs.append(agg3.reshape(N, D))

    return _final_stage(node_feat, aggs, Wpd, bpd, Wpe, bpe, Wp, bp)


# speedup vs baseline: 3.0048x; 1.1423x over previous
import functools

import jax
import jax.numpy as jnp
from jax import lax
from jax.experimental import pallas as pl
from jax.experimental.pallas import tpu as pltpu
from jax.experimental.pallas import tpu_sc as plsc

N = 10000
E = 160000
D = 256
DE = 16
H = 256

K = 5
EC = E // K

NODE_BLK = 1000
EDGE_BLK = 1600

NC = 2
NS = 16
NW = NC * NS
GB = 200
SB = 200
HH = H // 2
NSTRIPE = N // NS


def _pack_bf16_pair(lo, hi):
    ulo = lax.bitcast_convert_type(lo, jnp.uint32)
    uhi = lax.bitcast_convert_type(hi, jnp.uint32)
    rlo = ulo + jnp.uint32(0x7FFF) + ((ulo >> jnp.uint32(16)) & jnp.uint32(1))
    rhi = uhi + jnp.uint32(0x7FFF) + ((uhi >> jnp.uint32(16)) & jnp.uint32(1))
    packed = (rhi & jnp.uint32(0xFFFF0000)) | (rlo >> jnp.uint32(16))
    return lax.bitcast_convert_type(packed, jnp.float32)


def _unpack_bf16_pair(w):
    u = lax.bitcast_convert_type(w, jnp.uint32)
    lo = lax.bitcast_convert_type(u << jnp.uint32(16), jnp.float32)
    hi = lax.bitcast_convert_type(u & jnp.uint32(0xFFFF0000), jnp.float32)
    return lo, hi


def _node_pre_body(nf_ref, ws_ref, bs_ref, wd_ref, bd_ref, s_ref, d_ref):
    nf = nf_ref[...]
    s = jnp.dot(nf, ws_ref[...], preferred_element_type=jnp.float32) + bs_ref[...]
    d = jnp.dot(nf, wd_ref[...], preferred_element_type=jnp.float32) + bd_ref[...]
    s_ref[...] = _pack_bf16_pair(s[:, :HH], s[:, HH:])
    d_ref[...] = _pack_bf16_pair(d[:, :HH], d[:, HH:])


def _node_precompute(node_feat, Ws, bs, Wd, bd):
    return pl.pallas_call(
        _node_pre_body,
        grid=(N // NODE_BLK,),
        in_specs=[
            pl.BlockSpec((NODE_BLK, D), lambda i: (i, 0)),
            pl.BlockSpec((D, H), lambda i: (0, 0)),
            pl.BlockSpec((H,), lambda i: (0,)),
            pl.BlockSpec((D, H), lambda i: (0, 0)),
            pl.BlockSpec((H,), lambda i: (0,)),
        ],
        out_specs=[
            pl.BlockSpec((NODE_BLK, HH), lambda i: (i, 0)),
            pl.BlockSpec((NODE_BLK, HH), lambda i: (i, 0)),
        ],
        out_shape=[
            jax.ShapeDtypeStruct((N, HH), jnp.float32),
            jax.ShapeDtypeStruct((N, HH), jnp.float32),
        ],
    )(node_feat, Ws, bs, Wd, bd)


def _sc_gather(s, d, src_c, dst_c):
    per_w = EC // NW
    n_it = per_w // GB

    mesh = plsc.VectorSubcoreMesh(core_axis_name="c", subcore_axis_name="s")

    @functools.partial(
        pl.kernel,
        mesh=mesh,
        out_type=[
            jax.ShapeDtypeStruct((EC, HH), jnp.float32),
            jax.ShapeDtypeStruct((EC, HH), jnp.float32),
        ],
        scratch_types=[
            pltpu.VMEM((GB,), jnp.int32),
            pltpu.VMEM((GB,), jnp.int32),
            pltpu.VMEM((GB, HH), jnp.float32),
            pltpu.VMEM((GB, HH), jnp.float32),
            pltpu.SemaphoreType.DMA,
            pltpu.SemaphoreType.DMA,
        ],
    )
    def k(s_hbm, d_hbm, src_hbm, dst_hbm, gs_hbm, gd_hbm,
          idx_s, idx_d, rows_s, rows_d, sem_s, sem_d):
        wid = lax.axis_index("s") * NC + lax.axis_index("c")
        base = wid * per_w

        def body(i, carry):
            off = base + i * GB
            pltpu.sync_copy(src_hbm.at[pl.ds(off, GB)], idx_s)
            pltpu.sync_copy(dst_hbm.at[pl.ds(off, GB)], idx_d)
            cp_s = pltpu.async_copy(s_hbm.at[idx_s], rows_s, sem_s)
            cp_d = pltpu.async_copy(d_hbm.at[idx_d], rows_d, sem_d)
            cp_s.wait()
            cp_d.wait()
            pltpu.sync_copy(rows_s, gs_hbm.at[pl.ds(off, GB)])
            pltpu.sync_copy(rows_d, gd_hbm.at[pl.ds(off, GB)])
            return carry

        lax.fori_loop(0, n_it, body, 0)

    return k(s, d, src_c, dst_c)


def _edge_mlp_body(ea_ref, gs_ref, gd_ref, we1_ref, be1_ref, we2_ref, be2_ref,
                   wt1_ref, bt1_ref, wt2_ref, bt2_ref, m_ref):
    h1 = jnp.maximum(jnp.dot(ea_ref[...], we1_ref[...],
                             preferred_element_type=jnp.float32) + be1_ref[...], 0.0)
    ec = jnp.dot(h1.astype(jnp.bfloat16), we2_ref[...],
                 preferred_element_type=jnp.float32) + be2_ref[...]
    gs_lo, gs_hi = _unpack_bf16_pair(gs_ref[...])
    gd_lo, gd_hi = _unpack_bf16_pair(gd_ref[...])
    m1_lo = jnp.maximum(ec[:, :HH] + gs_lo + gd_lo, 0.0)
    m1_hi = jnp.maximum(ec[:, HH:] + gs_hi + gd_hi, 0.0)
    m1 = jnp.concatenate([m1_lo, m1_hi], axis=1)
    m2 = jnp.maximum(jnp.dot(m1.astype(jnp.bfloat16), wt1_ref[...],
                             preferred_element_type=jnp.float32) + bt1_ref[...], 0.0)
    m_ref[...] = jnp.dot(m2.astype(jnp.bfloat16), wt2_ref[...],
                         preferred_element_type=jnp.float32) + bt2_ref[...]


def _edge_mlp(edge_attr_c, gs, gd, We1, be1, We2b, be2, Wt1b, bt1, Wt2b, bt2):
    return pl.pallas_call(
        _edge_mlp_body,
        grid=(EC // EDGE_BLK,),
        in_specs=[
            pl.BlockSpec((EDGE_BLK, DE), lambda i: (i, 0)),
            pl.BlockSpec((EDGE_BLK, HH), lambda i: (i, 0)),
            pl.BlockSpec((EDGE_BLK, HH), lambda i: (i, 0)),
            pl.BlockSpec((DE, H), lambda i: (0, 0)),
            pl.BlockSpec((H,), lambda i: (0,)),
            pl.BlockSpec((H, H), lambda i: (0, 0)),
            pl.BlockSpec((H,), lambda i: (0,)),
            pl.BlockSpec((H, H), lambda i: (0, 0)),
            pl.BlockSpec((H,), lambda i: (0,)),
            pl.BlockSpec((H, D), lambda i: (0, 0)),
            pl.BlockSpec((D,), lambda i: (0,)),
        ],
        out_specs=pl.BlockSpec((EDGE_BLK, D), lambda i: (i, 0)),
        out_shape=jax.ShapeDtypeStruct((EC, D), jnp.float32),
    )(edge_attr_c, gs, gd, We1, be1, We2b, be2, Wt1b, bt1, Wt2b, bt2)


def _sc_scatter_add(m_chunks, dst, zeros_stripe):
    per_t = EC // NS
    n_it = per_t // SB

    mesh = plsc.VectorSubcoreMesh(core_axis_name="c", subcore_axis_name="s")

    @functools.partial(
        pl.kernel,
        mesh=mesh,
        out_type=jax.ShapeDtypeStruct((N, NC, HH), jnp.float32),
        scratch_types=[
            pltpu.VMEM((SB,), jnp.int32),
            pltpu.VMEM((SB, HH), jnp.float32),
            pltpu.VMEM_SHARED((N, HH), jnp.float32),
        ],
    )
    def k(m0, m1, m2, m3, m4, dst_hbm, z_hbm, out_hbm, idx_v, blk_v, acc):
        c = lax.axis_index("c")
        t = lax.axis_index("s")

        pltpu.sync_copy(z_hbm, acc.at[pl.ds(t * NSTRIPE, NSTRIPE)])
        plsc.subcore_barrier()

        for kk, m_hbm in enumerate((m0, m1, m2, m3, m4)):
            def body(i, carry, m_hbm=m_hbm, kk=kk):
                off = t * per_t + i * SB
                pltpu.sync_copy(dst_hbm.at[pl.ds(kk * EC + off, SB)], idx_v)
                pltpu.sync_copy(m_hbm.at[pl.ds(off, SB), c], blk_v)
                pltpu.sync_copy(blk_v, acc.at[idx_v], add=True)
                return carry

            lax.fori_loop(0, n_it, body, 0)

        plsc.subcore_barrier()

        pltpu.sync_copy(acc.at[pl.ds(t * NSTRIPE, NSTRIPE)],
                        out_hbm.at[pl.ds(t * NSTRIPE, NSTRIPE), c])

    return k(*m_chunks, dst, zeros_stripe)


def _final_body(nf_ref, agg_ref, wpd_ref, bpd_ref, wpe_ref, bpe_ref,
                wp_ref, bp_ref, out_ref):
    z = (jnp.dot(nf_ref[...], wpd_ref[...], preferred_element_type=jnp.float32)
         + bpd_ref[...]
         + jnp.dot(agg_ref[...], wpe_ref[...], preferred_element_type=jnp.float32)
         + bpe_ref[...])
    out_ref[...] = jnp.dot(jnp.maximum(z, 0.0), wp_ref[...],
                           preferred_element_type=jnp.float32) + bp_ref[...]


def _final_stage(node_feat, agg, Wpd, bpd, Wpe, bpe, Wp, bp):
    return pl.pallas_call(
        _final_body,
        grid=(N // NODE_BLK,),
        in_specs=[
            pl.BlockSpec((NODE_BLK, D), lambda i: (i, 0)),
            pl.BlockSpec((NODE_BLK, D), lambda i: (i, 0)),
            pl.BlockSpec((D, H), lambda i: (0, 0)),
            pl.BlockSpec((H,), lambda i: (0,)),
            pl.BlockSpec((D, H), lambda i: (0, 0)),
            pl.BlockSpec((H,), lambda i: (0,)),
            pl.BlockSpec((H, D), lambda i: (0, 0)),
            pl.BlockSpec((D,), lambda i: (0,)),
        ],
        out_specs=pl.BlockSpec((NODE_BLK, D), lambda i: (i, 0)),
        out_shape=jax.ShapeDtypeStruct((N, D), jnp.float32),
    )(node_feat, agg, Wpd, bpd, Wpe, bpe, Wp, bp)


def kernel(node_feat, edge_index, edge_attr, We1, be1, We2, be2, Ws, bs, Wd, bd,
           Wt1, bt1, Wt2, bt2, Wpd, bpd, Wpe, bpe, Wp, bp):
    src = edge_index[0]
    dst = edge_index[1]
    s, d = _node_precompute(node_feat, Ws, bs, Wd, bd)
    We2b = We2.astype(jnp.bfloat16)
    Wt1b = Wt1.astype(jnp.bfloat16)
    Wt2b = Wt2.astype(jnp.bfloat16)
    zeros_stripe = jnp.zeros((NSTRIPE, HH), jnp.float32)

    m_chunks = []
    for kk in range(K):
        lo = kk * EC
        src_c = lax.slice(src, (lo,), (lo + EC,))
        dst_c = lax.slice(dst, (lo,), (lo + EC,))
        ea_c = lax.slice(edge_attr, (lo, 0), (lo + EC, DE))
        gs, gd = _sc_gather(s, d, src_c, dst_c)
        m = _edge_mlp(ea_c, gs, gd, We1, be1, We2b, be2, Wt1b, bt1, Wt2b, bt2)
        m_chunks.append(m.reshape(EC, NC, HH))

    agg3 = _sc_scatter_add(m_chunks, dst, zeros_stripe)
    agg = agg3.reshape(N, D)
    return _final_stage(node_feat, agg, Wpd, bpd, Wpe, bpe, Wp, bp)


# speedup vs baseline: 4.5231x; 1.5053x over previous
import functools

import jax
import jax.numpy as jnp
from jax import lax
from jax.experimental import pallas as pl
from jax.experimental.pallas import tpu as pltpu
from jax.experimental.pallas import tpu_sc as plsc

N = 10000
E = 160000
D = 256
DE = 16
H = 256

K = 5
EC = E // K

NODE_BLK = 1000
EDGE_BLK = 1600

NC = 2
NS = 16
NW = NC * NS
GB = 200
SB = 200
HH = H // 2
NSTRIPE = N // NS

SCATTER_SPLIT = 3


def _pack_bf16_pair(lo, hi):
    ulo = lax.bitcast_convert_type(lo, jnp.uint32)
    uhi = lax.bitcast_convert_type(hi, jnp.uint32)
    rlo = ulo + jnp.uint32(0x7FFF) + ((ulo >> jnp.uint32(16)) & jnp.uint32(1))
    rhi = uhi + jnp.uint32(0x7FFF) + ((uhi >> jnp.uint32(16)) & jnp.uint32(1))
    packed = (rhi & jnp.uint32(0xFFFF0000)) | (rlo >> jnp.uint32(16))
    return lax.bitcast_convert_type(packed, jnp.float32)


def _unpack_bf16_pair(w):
    u = lax.bitcast_convert_type(w, jnp.uint32)
    lo = lax.bitcast_convert_type(u << jnp.uint32(16), jnp.float32)
    hi = lax.bitcast_convert_type(u & jnp.uint32(0xFFFF0000), jnp.float32)
    return lo, hi


def _node_pre_body(nf_ref, ws_ref, bs_ref, wd_ref, bd_ref, s_ref, d_ref):
    nf = nf_ref[...]
    s = jnp.dot(nf, ws_ref[...], preferred_element_type=jnp.float32) + bs_ref[...]
    d = jnp.dot(nf, wd_ref[...], preferred_element_type=jnp.float32) + bd_ref[...]
    s_ref[...] = _pack_bf16_pair(s[:, :HH], s[:, HH:])
    d_ref[...] = _pack_bf16_pair(d[:, :HH], d[:, HH:])


def _node_precompute(node_feat, Ws, bs, Wd, bd):
    return pl.pallas_call(
        _node_pre_body,
        grid=(N // NODE_BLK,),
        in_specs=[
            pl.BlockSpec((NODE_BLK, D), lambda i: (i, 0)),
            pl.BlockSpec((D, H), lambda i: (0, 0)),
            pl.BlockSpec((H,), lambda i: (0,)),
            pl.BlockSpec((D, H), lambda i: (0, 0)),
            pl.BlockSpec((H,), lambda i: (0,)),
        ],
        out_specs=[
            pl.BlockSpec((NODE_BLK, HH), lambda i: (i, 0)),
            pl.BlockSpec((NODE_BLK, HH), lambda i: (i, 0)),
        ],
        out_shape=[
            jax.ShapeDtypeStruct((N, HH), jnp.float32),
            jax.ShapeDtypeStruct((N, HH), jnp.float32),
        ],
    )(node_feat, Ws, bs, Wd, bd)


def _sc_gather(s, d, src, dst, kk):
    per_w = EC // NW
    n_it = per_w // GB

    mesh = plsc.VectorSubcoreMesh(core_axis_name="c", subcore_axis_name="s")

    @functools.partial(
        pl.kernel,
        mesh=mesh,
        out_type=[
            jax.ShapeDtypeStruct((EC, HH), jnp.float32),
            jax.ShapeDtypeStruct((EC, HH), jnp.float32),
        ],
        scratch_types=[
            pltpu.VMEM((GB,), jnp.int32),
            pltpu.VMEM((GB,), jnp.int32),
            pltpu.VMEM((GB, HH), jnp.float32),
            pltpu.VMEM((GB, HH), jnp.float32),
            pltpu.SemaphoreType.DMA,
            pltpu.SemaphoreType.DMA,
        ],
    )
    def k(s_hbm, d_hbm, src_hbm, dst_hbm, gs_hbm, gd_hbm,
          idx_s, idx_d, rows_s, rows_d, sem_s, sem_d):
        wid = lax.axis_index("s") * NC + lax.axis_index("c")
        base = wid * per_w

        def body(i, carry):
            off = base + i * GB
            pltpu.sync_copy(src_hbm.at[pl.ds(kk * EC + off, GB)], idx_s)
            pltpu.sync_copy(dst_hbm.at[pl.ds(kk * EC + off, GB)], idx_d)
            cp_s = pltpu.async_copy(s_hbm.at[idx_s], rows_s, sem_s)
            cp_d = pltpu.async_copy(d_hbm.at[idx_d], rows_d, sem_d)
            cp_s.wait()
            cp_d.wait()
            pltpu.sync_copy(rows_s, gs_hbm.at[pl.ds(off, GB)])
            pltpu.sync_copy(rows_d, gd_hbm.at[pl.ds(off, GB)])
            return carry

        lax.fori_loop(0, n_it, body, 0)

    return k(s, d, src, dst)


def _edge_mlp_body(ea_ref, gs_ref, gd_ref, we1_ref, be1_ref, we2_ref, be2_ref,
                   wt1_ref, bt1_ref, wt2_ref, bt2_ref, m_ref):
    h1 = jnp.maximum(jnp.dot(ea_ref[...], we1_ref[...],
                             preferred_element_type=jnp.float32) + be1_ref[...], 0.0)
    ec = jnp.dot(h1.astype(jnp.bfloat16), we2_ref[...],
                 preferred_element_type=jnp.float32) + be2_ref[...]
    gs_lo, gs_hi = _unpack_bf16_pair(gs_ref[...])
    gd_lo, gd_hi = _unpack_bf16_pair(gd_ref[...])
    m1_lo = jnp.maximum(ec[:, :HH] + gs_lo + gd_lo, 0.0)
    m1_hi = jnp.maximum(ec[:, HH:] + gs_hi + gd_hi, 0.0)
    m1 = jnp.concatenate([m1_lo, m1_hi], axis=1)
    m2 = jnp.maximum(jnp.dot(m1.astype(jnp.bfloat16), wt1_ref[...],
                             preferred_element_type=jnp.float32) + bt1_ref[...], 0.0)
    m_ref[...] = jnp.dot(m2.astype(jnp.bfloat16), wt2_ref[...],
                         preferred_element_type=jnp.float32) + bt2_ref[...]


def _edge_mlp(edge_attr, gs, gd, We1, be1, We2b, be2, Wt1b, bt1, Wt2b, bt2, kk):
    nb = EC // EDGE_BLK
    return pl.pallas_call(
        _edge_mlp_body,
        grid=(nb,),
        in_specs=[
            pl.BlockSpec((EDGE_BLK, DE), lambda i: (i + kk * nb, 0)),
            pl.BlockSpec((EDGE_BLK, HH), lambda i: (i, 0)),
            pl.BlockSpec((EDGE_BLK, HH), lambda i: (i, 0)),
            pl.BlockSpec((DE, H), lambda i: (0, 0)),
            pl.BlockSpec((H,), lambda i: (0,)),
            pl.BlockSpec((H, H), lambda i: (0, 0)),
            pl.BlockSpec((H,), lambda i: (0,)),
            pl.BlockSpec((H, H), lambda i: (0, 0)),
            pl.BlockSpec((H,), lambda i: (0,)),
            pl.BlockSpec((H, D), lambda i: (0, 0)),
            pl.BlockSpec((D,), lambda i: (0,)),
        ],
        out_specs=pl.BlockSpec((EDGE_BLK, D), lambda i: (i, 0)),
        out_shape=jax.ShapeDtypeStruct((EC, D), jnp.float32),
    )(edge_attr, gs, gd, We1, be1, We2b, be2, Wt1b, bt1, Wt2b, bt2)


def _sc_scatter_add(m_chunks, chunk_ids, dst, zeros_stripe):
    per_t = EC // NS
    n_it = per_t // SB
    nchunks = len(m_chunks)

    mesh = plsc.VectorSubcoreMesh(core_axis_name="c", subcore_axis_name="s")

    @functools.partial(
        pl.kernel,
        mesh=mesh,
        out_type=jax.ShapeDtypeStruct((N, NC, HH), jnp.float32),
        scratch_types=[
            pltpu.VMEM((SB,), jnp.int32),
            pltpu.VMEM((SB, HH), jnp.float32),
            pltpu.VMEM_SHARED((N, HH), jnp.float32),
        ],
    )
    def k(*refs):
        m_hbms = refs[:nchunks]
        dst_hbm, z_hbm, out_hbm, idx_v, blk_v, acc = refs[nchunks:]
        c = lax.axis_index("c")
        t = lax.axis_index("s")
        col = c * HH

        pltpu.sync_copy(z_hbm, acc.at[pl.ds(t * NSTRIPE, NSTRIPE)])
        plsc.subcore_barrier()

        for j, m_hbm in enumerate(m_hbms):
            kk = chunk_ids[j]

            def body(i, carry, m_hbm=m_hbm, kk=kk):
                off = t * per_t + i * SB
                pltpu.sync_copy(dst_hbm.at[pl.ds(kk * EC + off, SB)], idx_v)
                pltpu.sync_copy(m_hbm.at[pl.ds(off, SB), pl.ds(col, HH)], blk_v)
                pltpu.sync_copy(blk_v, acc.at[idx_v], add=True)
                return carry

            lax.fori_loop(0, n_it, body, 0)

        plsc.subcore_barrier()

        pltpu.sync_copy(acc.at[pl.ds(t * NSTRIPE, NSTRIPE)],
                        out_hbm.at[pl.ds(t * NSTRIPE, NSTRIPE), c])

    return k(*m_chunks, dst, zeros_stripe)


def _final_body(nf_ref, agg_a_ref, agg_b_ref, wpd_ref, bpd_ref, wpe_ref,
                bpe_ref, wp_ref, bp_ref, out_ref):
    a = agg_a_ref[...] + agg_b_ref[...]
    agg = jnp.concatenate([a[:, 0, :], a[:, 1, :]], axis=1)
    z = (jnp.dot(nf_ref[...], wpd_ref[...], preferred_element_type=jnp.float32)
         + bpd_ref[...]
         + jnp.dot(agg, wpe_ref[...], preferred_element_type=jnp.float32)
         + bpe_ref[...])
    out_ref[...] = jnp.dot(jnp.maximum(z, 0.0), wp_ref[...],
                           preferred_element_type=jnp.float32) + bp_ref[...]


def _final_stage(node_feat, agg_a, agg_b, Wpd, bpd, Wpe, bpe, Wp, bp):
    return pl.pallas_call(
        _final_body,
        grid=(N // NODE_BLK,),
        in_specs=[
            pl.BlockSpec((NODE_BLK, D), lambda i: (i, 0)),
            pl.BlockSpec((NODE_BLK, NC, HH), lambda i: (i, 0, 0)),
            pl.BlockSpec((NODE_BLK, NC, HH), lambda i: (i, 0, 0)),
            pl.BlockSpec((D, H), lambda i: (0, 0)),
            pl.BlockSpec((H,), lambda i: (0,)),
            pl.BlockSpec((D, H), lambda i: (0, 0)),
            pl.BlockSpec((H,), lambda i: (0,)),
            pl.BlockSpec((H, D), lambda i: (0, 0)),
            pl.BlockSpec((D,), lambda i: (0,)),
        ],
        out_specs=pl.BlockSpec((NODE_BLK, D), lambda i: (i, 0)),
        out_shape=jax.ShapeDtypeStruct((N, D), jnp.float32),
    )(node_feat, agg_a, agg_b, Wpd, bpd, Wpe, bpe, Wp, bp)


def kernel(node_feat, edge_index, edge_attr, We1, be1, We2, be2, Ws, bs, Wd, bd,
           Wt1, bt1, Wt2, bt2, Wpd, bpd, Wpe, bpe, Wp, bp):
    src = edge_index[0]
    dst = edge_index[1]
    s, d = _node_precompute(node_feat, Ws, bs, Wd, bd)
    We2b = We2.astype(jnp.bfloat16)
    Wt1b = Wt1.astype(jnp.bfloat16)
    Wt2b = Wt2.astype(jnp.bfloat16)
    zeros_stripe = jnp.zeros((NSTRIPE, HH), jnp.float32)

    m_chunks = []
    for kk in range(K):
        gs, gd = _sc_gather(s, d, src, dst, kk)
        m = _edge_mlp(edge_attr, gs, gd, We1, be1, We2b, be2, Wt1b, bt1,
                      Wt2b, bt2, kk)
        m_chunks.append(m)

    agg_a = _sc_scatter_add(m_chunks[:SCATTER_SPLIT],
                            list(range(SCATTER_SPLIT)), dst, zeros_stripe)
    agg_b = _sc_scatter_add(m_chunks[SCATTER_SPLIT:],
                            list(range(SCATTER_SPLIT, K)), dst, zeros_stripe)
    return _final_stage(node_feat, agg_a, agg_b, Wpd, bpd, Wpe, bpe, Wp, bp)


# speedup vs baseline: 4.8367x; 1.0693x over previous
import functools

import jax
import jax.numpy as jnp
from jax import lax
from jax.experimental import pallas as pl
from jax.experimental.pallas import tpu as pltpu
from jax.experimental.pallas import tpu_sc as plsc

N = 10000
E = 160000
D = 256
DE = 16
H = 256

K = 5
EC = E // K

NODE_BLK = 1000
EDGE_BLK = 1280

NC = 2
NS = 16
NW = NC * NS
GB = 200
SB = 200
HH = H // 2
NSTRIPE = N // NS

SCATTER_SPLIT = 3


def _pack_bf16_pair(lo, hi):
    ulo = lax.bitcast_convert_type(lo, jnp.uint32)
    uhi = lax.bitcast_convert_type(hi, jnp.uint32)
    rlo = ulo + jnp.uint32(0x7FFF) + ((ulo >> jnp.uint32(16)) & jnp.uint32(1))
    rhi = uhi + jnp.uint32(0x7FFF) + ((uhi >> jnp.uint32(16)) & jnp.uint32(1))
    packed = (rhi & jnp.uint32(0xFFFF0000)) | (rlo >> jnp.uint32(16))
    return lax.bitcast_convert_type(packed, jnp.float32)


def _unpack_bf16_pair(w):
    u = lax.bitcast_convert_type(w, jnp.uint32)
    lo = lax.bitcast_convert_type(u << jnp.uint32(16), jnp.float32)
    hi = lax.bitcast_convert_type(u & jnp.uint32(0xFFFF0000), jnp.float32)
    return lo, hi


def _node_pre_body(nf_ref, ws_ref, bs_ref, wd_ref, bd_ref, s_ref, d_ref):
    nf = nf_ref[...]
    s = jnp.dot(nf, ws_ref[...], preferred_element_type=jnp.float32) + bs_ref[...]
    d = jnp.dot(nf, wd_ref[...], preferred_element_type=jnp.float32) + bd_ref[...]
    s_ref[...] = _pack_bf16_pair(s[:, :HH], s[:, HH:])
    d_ref[...] = _pack_bf16_pair(d[:, :HH], d[:, HH:])


def _node_precompute(node_feat, Ws, bs, Wd, bd):
    return pl.pallas_call(
        _node_pre_body,
        grid=(N // NODE_BLK,),
        in_specs=[
            pl.BlockSpec((NODE_BLK, D), lambda i: (i, 0)),
            pl.BlockSpec((D, H), lambda i: (0, 0)),
            pl.BlockSpec((H,), lambda i: (0,)),
            pl.BlockSpec((D, H), lambda i: (0, 0)),
            pl.BlockSpec((H,), lambda i: (0,)),
        ],
        out_specs=[
            pl.BlockSpec((NODE_BLK, HH), lambda i: (i, 0)),
            pl.BlockSpec((NODE_BLK, HH), lambda i: (i, 0)),
        ],
        out_shape=[
            jax.ShapeDtypeStruct((N, HH), jnp.float32),
            jax.ShapeDtypeStruct((N, HH), jnp.float32),
        ],
    )(node_feat, Ws, bs, Wd, bd)


def _sc_gather(s, d, src, dst, kk):
    per_w = EC // NW
    n_it = per_w // GB

    mesh = plsc.VectorSubcoreMesh(core_axis_name="c", subcore_axis_name="s")

    @functools.partial(
        pl.kernel,
        mesh=mesh,
        out_type=[
            jax.ShapeDtypeStruct((EC, HH), jnp.float32),
            jax.ShapeDtypeStruct((EC, HH), jnp.float32),
        ],
        scratch_types=[
            pltpu.VMEM((GB,), jnp.int32),
            pltpu.VMEM((GB,), jnp.int32),
            pltpu.VMEM((GB, HH), jnp.float32),
            pltpu.VMEM((GB, HH), jnp.float32),
            pltpu.SemaphoreType.DMA,
            pltpu.SemaphoreType.DMA,
        ],
    )
    def k(s_hbm, d_hbm, src_hbm, dst_hbm, gs_hbm, gd_hbm,
          idx_s, idx_d, rows_s, rows_d, sem_s, sem_d):
        wid = lax.axis_index("s") * NC + lax.axis_index("c")
        base = wid * per_w

        def body(i, carry):
            off = base + i * GB
            pltpu.sync_copy(src_hbm.at[pl.ds(kk * EC + off, GB)], idx_s)
            pltpu.sync_copy(dst_hbm.at[pl.ds(kk * EC + off, GB)], idx_d)
            cp_s = pltpu.async_copy(s_hbm.at[idx_s], rows_s, sem_s)
            cp_d = pltpu.async_copy(d_hbm.at[idx_d], rows_d, sem_d)
            cp_s.wait()
            cp_d.wait()
            pltpu.sync_copy(rows_s, gs_hbm.at[pl.ds(off, GB)])
            pltpu.sync_copy(rows_d, gd_hbm.at[pl.ds(off, GB)])
            return carry

        lax.fori_loop(0, n_it, body, 0)

    return k(s, d, src, dst)


def _edge_mlp_body(ea_ref, gs_ref, gd_ref, we1_ref, be1_ref, we2_ref, be2_ref,
                   wt1_ref, bt1_ref, wt2_ref, bt2_ref, m_ref):
    h1 = jnp.maximum(
        lax.dot_general(ea_ref[...], we1_ref[...], (((0,), (0,)), ((), ())),
                        preferred_element_type=jnp.float32) + be1_ref[...], 0.0)
    ec = jnp.dot(h1.astype(jnp.bfloat16), we2_ref[...],
                 preferred_element_type=jnp.float32) + be2_ref[...]
    gs_lo, gs_hi = _unpack_bf16_pair(gs_ref[...])
    gd_lo, gd_hi = _unpack_bf16_pair(gd_ref[...])
    m1_lo = jnp.maximum(ec[:, :HH] + gs_lo + gd_lo, 0.0)
    m1_hi = jnp.maximum(ec[:, HH:] + gs_hi + gd_hi, 0.0)
    m1 = jnp.concatenate([m1_lo, m1_hi], axis=1)
    m2 = jnp.maximum(jnp.dot(m1.astype(jnp.bfloat16), wt1_ref[...],
                             preferred_element_type=jnp.float32) + bt1_ref[...], 0.0)
    m_ref[...] = jnp.dot(m2.astype(jnp.bfloat16), wt2_ref[...],
                         preferred_element_type=jnp.float32) + bt2_ref[...]


def _edge_mlp(edge_attr, gs, gd, We1, be1, We2b, be2, Wt1b, bt1, Wt2b, bt2, kk):
    nb = EC // EDGE_BLK
    return pl.pallas_call(
        _edge_mlp_body,
        grid=(nb,),
        in_specs=[
            pl.BlockSpec((DE, EDGE_BLK), lambda i, kk=kk: (0, i + kk * nb)),
            pl.BlockSpec((EDGE_BLK, HH), lambda i: (i, 0)),
            pl.BlockSpec((EDGE_BLK, HH), lambda i: (i, 0)),
            pl.BlockSpec((DE, H), lambda i: (0, 0)),
            pl.BlockSpec((H,), lambda i: (0,)),
            pl.BlockSpec((H, H), lambda i: (0, 0)),
            pl.BlockSpec((H,), lambda i: (0,)),
            pl.BlockSpec((H, H), lambda i: (0, 0)),
            pl.BlockSpec((H,), lambda i: (0,)),
            pl.BlockSpec((H, D), lambda i: (0, 0)),
            pl.BlockSpec((D,), lambda i: (0,)),
        ],
        out_specs=pl.BlockSpec((EDGE_BLK, D), lambda i: (i, 0)),
        out_shape=jax.ShapeDtypeStruct((EC, D), jnp.float32),
    )(edge_attr, gs, gd, We1, be1, We2b, be2, Wt1b, bt1, Wt2b, bt2)


def _sc_scatter_add(m_chunks, chunk_ids, dst, zeros_stripe):
    per_t = EC // NS
    n_it = per_t // SB
    nchunks = len(m_chunks)

    mesh = plsc.VectorSubcoreMesh(core_axis_name="c", subcore_axis_name="s")

    @functools.partial(
        pl.kernel,
        mesh=mesh,
        out_type=jax.ShapeDtypeStruct((N, NC, HH), jnp.float32),
        scratch_types=[
            pltpu.VMEM((SB,), jnp.int32),
            pltpu.VMEM((SB, HH), jnp.float32),
            pltpu.VMEM_SHARED((N, HH), jnp.float32),
        ],
    )
    def k(*refs):
        m_hbms = refs[:nchunks]
        dst_hbm, z_hbm, out_hbm, idx_v, blk_v, acc = refs[nchunks:]
        c = lax.axis_index("c")
        t = lax.axis_index("s")
        col = c * HH

        pltpu.sync_copy(z_hbm, acc.at[pl.ds(t * NSTRIPE, NSTRIPE)])
        plsc.subcore_barrier()

        for j, m_hbm in enumerate(m_hbms):
            kk = chunk_ids[j]

            def body(i, carry, m_hbm=m_hbm, kk=kk):
                off = t * per_t + i * SB
                pltpu.sync_copy(dst_hbm.at[pl.ds(kk * EC + off, SB)], idx_v)
                pltpu.sync_copy(m_hbm.at[pl.ds(off, SB), pl.ds(col, HH)], blk_v)
                pltpu.sync_copy(blk_v, acc.at[idx_v], add=True)
                return carry

            lax.fori_loop(0, n_it, body, 0)

        plsc.subcore_barrier()

        pltpu.sync_copy(acc.at[pl.ds(t * NSTRIPE, NSTRIPE)],
                        out_hbm.at[pl.ds(t * NSTRIPE, NSTRIPE), c])

    return k(*m_chunks, dst, zeros_stripe)


def _final_body(nf_ref, agg_a_ref, agg_b_ref, wpd_ref, bpd_ref, wpe_ref,
                bpe_ref, wp_ref, bp_ref, out_ref):
    a = agg_a_ref[...] + agg_b_ref[...]
    agg = jnp.concatenate([a[:, 0, :], a[:, 1, :]], axis=1)
    z = (jnp.dot(nf_ref[...], wpd_ref[...], preferred_element_type=jnp.float32)
         + bpd_ref[...]
         + jnp.dot(agg, wpe_ref[...], preferred_element_type=jnp.float32)
         + bpe_ref[...])
    out_ref[...] = jnp.dot(jnp.maximum(z, 0.0), wp_ref[...],
                           preferred_element_type=jnp.float32) + bp_ref[...]


def _final_stage(node_feat, agg_a, agg_b, Wpd, bpd, Wpe, bpe, Wp, bp):
    return pl.pallas_call(
        _final_body,
        grid=(N // NODE_BLK,),
        in_specs=[
            pl.BlockSpec((NODE_BLK, D), lambda i: (i, 0)),
            pl.BlockSpec((NODE_BLK, NC, HH), lambda i: (i, 0, 0)),
            pl.BlockSpec((NODE_BLK, NC, HH), lambda i: (i, 0, 0)),
            pl.BlockSpec((D, H), lambda i: (0, 0)),
            pl.BlockSpec((H,), lambda i: (0,)),
            pl.BlockSpec((D, H), lambda i: (0, 0)),
            pl.BlockSpec((H,), lambda i: (0,)),
            pl.BlockSpec((H, D), lambda i: (0, 0)),
            pl.BlockSpec((D,), lambda i: (0,)),
        ],
        out_specs=pl.BlockSpec((NODE_BLK, D), lambda i: (i, 0)),
        out_shape=jax.ShapeDtypeStruct((N, D), jnp.float32),
    )(node_feat, agg_a, agg_b, Wpd, bpd, Wpe, bpe, Wp, bp)


def kernel(node_feat, edge_index, edge_attr, We1, be1, We2, be2, Ws, bs, Wd, bd,
           Wt1, bt1, Wt2, bt2, Wpd, bpd, Wpe, bpe, Wp, bp):
    src = edge_index[0]
    dst = edge_index[1]
    s, d = _node_precompute(node_feat, Ws, bs, Wd, bd)
    We2b = We2.astype(jnp.bfloat16)
    Wt1b = Wt1.astype(jnp.bfloat16)
    Wt2b = Wt2.astype(jnp.bfloat16)
    zeros_stripe = jnp.zeros((NSTRIPE, HH), jnp.float32)

    ea_t = edge_attr.T
    m_chunks = []
    for kk in range(K):
        gs, gd = _sc_gather(s, d, src, dst, kk)
        m = _edge_mlp(ea_t, gs, gd, We1, be1, We2b, be2, Wt1b, bt1,
                      Wt2b, bt2, kk)
        m_chunks.append(m)

    agg_a = _sc_scatter_add(m_chunks[:SCATTER_SPLIT],
                            list(range(SCATTER_SPLIT)), dst, zeros_stripe)
    agg_b = _sc_scatter_add(m_chunks[SCATTER_SPLIT:],
                            list(range(SCATTER_SPLIT, K)), dst, zeros_stripe)
    return _final_stage(node_feat, agg_a, agg_b, Wpd, bpd, Wpe, bpe, Wp, bp)


# speedup vs baseline: 5.0230x; 1.0385x over previous
import functools

import jax
import jax.numpy as jnp
from jax import lax
from jax.experimental import pallas as pl
from jax.experimental.pallas import tpu as pltpu
from jax.experimental.pallas import tpu_sc as plsc

N = 10000
E = 160000
D = 256
DE = 16
H = 256

K = 5
EC = E // K

NODE_BLK = 1000
EDGE_BLK = 1280

NC = 2
NS = 16
NW = NC * NS
GB = 200
SB = 200
HH = H // 2
NSTRIPE = N // NS

SCATTER_SPLIT = 3


def _pack_bf16_pair(lo, hi):
    ulo = lax.bitcast_convert_type(lo, jnp.uint32)
    uhi = lax.bitcast_convert_type(hi, jnp.uint32)
    rlo = ulo + jnp.uint32(0x7FFF) + ((ulo >> jnp.uint32(16)) & jnp.uint32(1))
    rhi = uhi + jnp.uint32(0x7FFF) + ((uhi >> jnp.uint32(16)) & jnp.uint32(1))
    packed = (rhi & jnp.uint32(0xFFFF0000)) | (rlo >> jnp.uint32(16))
    return lax.bitcast_convert_type(packed, jnp.float32)


def _unpack_bf16_pair(w):
    u = lax.bitcast_convert_type(w, jnp.uint32)
    lo = lax.bitcast_convert_type(u << jnp.uint32(16), jnp.float32)
    hi = lax.bitcast_convert_type(u & jnp.uint32(0xFFFF0000), jnp.float32)
    return lo, hi


def _node_pre_body(nf_ref, ws_ref, bs_ref, wd_ref, bd_ref, s_ref, d_ref):
    nf = nf_ref[...]
    s = jnp.dot(nf, ws_ref[...], preferred_element_type=jnp.float32) + bs_ref[...]
    d = jnp.dot(nf, wd_ref[...], preferred_element_type=jnp.float32) + bd_ref[...]
    s_ref[...] = _pack_bf16_pair(s[:, :HH], s[:, HH:])
    d_ref[...] = _pack_bf16_pair(d[:, :HH], d[:, HH:])


def _node_precompute(node_feat, Ws, bs, Wd, bd):
    return pl.pallas_call(
        _node_pre_body,
        grid=(N // NODE_BLK,),
        in_specs=[
            pl.BlockSpec((NODE_BLK, D), lambda i: (i, 0)),
            pl.BlockSpec((D, H), lambda i: (0, 0)),
            pl.BlockSpec((H,), lambda i: (0,)),
            pl.BlockSpec((D, H), lambda i: (0, 0)),
            pl.BlockSpec((H,), lambda i: (0,)),
        ],
        out_specs=[
            pl.BlockSpec((NODE_BLK, HH), lambda i: (i, 0)),
            pl.BlockSpec((NODE_BLK, HH), lambda i: (i, 0)),
        ],
        out_shape=[
            jax.ShapeDtypeStruct((N, HH), jnp.float32),
            jax.ShapeDtypeStruct((N, HH), jnp.float32),
        ],
    )(node_feat, Ws, bs, Wd, bd)


def _sc_gather(s, d, src_r, dst_r, kk):
    per_w = EC // NW
    n_it = per_w // GB
    rows_per_chunk = EC // GB

    mesh = plsc.VectorSubcoreMesh(core_axis_name="c", subcore_axis_name="s")

    @functools.partial(
        pl.kernel,
        mesh=mesh,
        out_type=[
            jax.ShapeDtypeStruct((EC, HH), jnp.float32),
            jax.ShapeDtypeStruct((EC, HH), jnp.float32),
        ],
        scratch_types=[
            pltpu.VMEM((per_w,), jnp.int32),
            pltpu.VMEM((per_w,), jnp.int32),
            pltpu.VMEM((GB, HH), jnp.float32),
            pltpu.VMEM((GB, HH), jnp.float32),
            pltpu.VMEM((GB, HH), jnp.float32),
            pltpu.VMEM((GB, HH), jnp.float32),
        ]
        + [pltpu.SemaphoreType.DMA] * 8,
    )
    def k(s_hbm, d_hbm, src_hbm, dst_hbm, gs_hbm, gd_hbm,
          idx_s, idx_d, rs0, rs1, rd0, rd1,
          gs0, gs1, gd0, gd1, ws0, ws1, wd0, wd1):
        wid = lax.axis_index("s") * NC + lax.axis_index("c")
        base = wid * per_w

        pltpu.sync_copy(src_hbm.at[pl.ds(kk * EC + base, per_w)], idx_s)
        pltpu.sync_copy(dst_hbm.at[pl.ds(kk * EC + base, per_w)], idx_d)

        rows_s = (rs0, rs1)
        rows_d = (rd0, rd1)
        gsem_s = (gs0, gs1)
        gsem_d = (gd0, gd1)
        wsem_s = (ws0, ws1)
        wsem_d = (wd0, wd1)
        gh_s = [None] * n_it
        gh_d = [None] * n_it
        wh_s = [None] * n_it
        wh_d = [None] * n_it

        def start_gather(i):
            b = i & 1
            gh_s[i] = pltpu.async_copy(
                s_hbm.at[idx_s.at[pl.ds(i * GB, GB)]], rows_s[b], gsem_s[b])
            gh_d[i] = pltpu.async_copy(
                d_hbm.at[idx_d.at[pl.ds(i * GB, GB)]], rows_d[b], gsem_d[b])

        start_gather(0)
        for i in range(n_it):
            b = i & 1
            if i + 1 < n_it:
                if i >= 1:
                    wh_s[i - 1].wait()
                    wh_d[i - 1].wait()
                start_gather(i + 1)
            gh_s[i].wait()
            gh_d[i].wait()
            off = base + i * GB
            wh_s[i] = pltpu.async_copy(rows_s[b], gs_hbm.at[pl.ds(off, GB)], wsem_s[b])
            wh_d[i] = pltpu.async_copy(rows_d[b], gd_hbm.at[pl.ds(off, GB)], wsem_d[b])
        for i in range(max(0, n_it - 2), n_it):
            wh_s[i].wait()
            wh_d[i].wait()

    return k(s, d, src_r, dst_r)


def _edge_mlp_body(ea_ref, gs_ref, gd_ref, we1_ref, be1_ref, we2_ref, be2_ref,
                   wt1_ref, bt1_ref, wt2_ref, bt2_ref, m_ref):
    h1 = jnp.maximum(
        lax.dot_general(ea_ref[...], we1_ref[...], (((0,), (0,)), ((), ())),
                        preferred_element_type=jnp.float32) + be1_ref[...], 0.0)
    ec = jnp.dot(h1.astype(jnp.bfloat16), we2_ref[...],
                 preferred_element_type=jnp.float32) + be2_ref[...]
    gs_lo, gs_hi = _unpack_bf16_pair(gs_ref[...])
    gd_lo, gd_hi = _unpack_bf16_pair(gd_ref[...])
    m1_lo = jnp.maximum(ec[:, :HH] + gs_lo + gd_lo, 0.0)
    m1_hi = jnp.maximum(ec[:, HH:] + gs_hi + gd_hi, 0.0)
    m1 = jnp.concatenate([m1_lo, m1_hi], axis=1)
    m2 = jnp.maximum(jnp.dot(m1.astype(jnp.bfloat16), wt1_ref[...],
                             preferred_element_type=jnp.float32) + bt1_ref[...], 0.0)
    m_ref[...] = jnp.dot(m2.astype(jnp.bfloat16), wt2_ref[...],
                         preferred_element_type=jnp.float32) + bt2_ref[...]


def _edge_mlp(edge_attr, gs, gd, We1, be1, We2b, be2, Wt1b, bt1, Wt2b, bt2, kk):
    nb = EC // EDGE_BLK
    return pl.pallas_call(
        _edge_mlp_body,
        grid=(nb,),
        in_specs=[
            pl.BlockSpec((DE, EDGE_BLK), lambda i, kk=kk: (0, i + kk * nb)),
            pl.BlockSpec((EDGE_BLK, HH), lambda i: (i, 0)),
            pl.BlockSpec((EDGE_BLK, HH), lambda i: (i, 0)),
            pl.BlockSpec((DE, H), lambda i: (0, 0)),
            pl.BlockSpec((H,), lambda i: (0,)),
            pl.BlockSpec((H, H), lambda i: (0, 0)),
            pl.BlockSpec((H,), lambda i: (0,)),
            pl.BlockSpec((H, H), lambda i: (0, 0)),
            pl.BlockSpec((H,), lambda i: (0,)),
            pl.BlockSpec((H, D), lambda i: (0, 0)),
            pl.BlockSpec((D,), lambda i: (0,)),
        ],
        out_specs=pl.BlockSpec((EDGE_BLK, D), lambda i: (i, 0)),
        out_shape=jax.ShapeDtypeStruct((EC, D), jnp.float32),
    )(edge_attr, gs, gd, We1, be1, We2b, be2, Wt1b, bt1, Wt2b, bt2)


def _sc_scatter_add(m_chunks, chunk_ids, dst_r, zeros_stripe):
    per_t = EC // NS
    n_it = per_t // SB
    rows_per_chunk = EC // SB
    nchunks = len(m_chunks)

    mesh = plsc.VectorSubcoreMesh(core_axis_name="c", subcore_axis_name="s")

    @functools.partial(
        pl.kernel,
        mesh=mesh,
        out_type=jax.ShapeDtypeStruct((N, NC, HH), jnp.float32),
        scratch_types=[
            pltpu.VMEM((SB,), jnp.int32),
            pltpu.VMEM((SB, HH), jnp.float32),
            pltpu.VMEM_SHARED((N, HH), jnp.float32),
        ],
    )
    def k(*refs):
        m_hbms = refs[:nchunks]
        (dst_hbm, z_hbm, out_hbm, idx_v, blk_v, acc) = refs[nchunks:]
        c = lax.axis_index("c")
        t = lax.axis_index("s")
        col = c * HH

        pltpu.sync_copy(z_hbm, acc.at[pl.ds(t * NSTRIPE, NSTRIPE)])
        plsc.subcore_barrier()

        for j, m_hbm in enumerate(m_hbms):
            kk = chunk_ids[j]

            def body(i, carry, m_hbm=m_hbm, kk=kk):
                off = t * per_t + i * SB
                pltpu.sync_copy(dst_hbm.at[pl.ds(kk * EC + off, SB)], idx_v)
                pltpu.sync_copy(m_hbm.at[pl.ds(off, SB), pl.ds(col, HH)], blk_v)
                pltpu.sync_copy(blk_v, acc.at[idx_v], add=True)
                return carry

            lax.fori_loop(0, n_it, body, 0)

        plsc.subcore_barrier()

        pltpu.sync_copy(acc.at[pl.ds(t * NSTRIPE, NSTRIPE)],
                        out_hbm.at[pl.ds(t * NSTRIPE, NSTRIPE), c])

    return k(*m_chunks, dst_r, zeros_stripe)


def _final_body(nf_ref, agg_a_ref, agg_b_ref, wpd_ref, bpd_ref, wpe_ref,
                bpe_ref, wp_ref, bp_ref, out_ref):
    a = agg_a_ref[...] + agg_b_ref[...]
    agg = jnp.concatenate([a[:, 0, :], a[:, 1, :]], axis=1)
    z = (jnp.dot(nf_ref[...], wpd_ref[...], preferred_element_type=jnp.float32)
         + bpd_ref[...]
         + jnp.dot(agg, wpe_ref[...], preferred_element_type=jnp.float32)
         + bpe_ref[...])
    out_ref[...] = jnp.dot(jnp.maximum(z, 0.0), wp_ref[...],
                           preferred_element_type=jnp.float32) + bp_ref[...]


def _final_stage(node_feat, agg_a, agg_b, Wpd, bpd, Wpe, bpe, Wp, bp):
    return pl.pallas_call(
        _final_body,
        grid=(N // NODE_BLK,),
        in_specs=[
            pl.BlockSpec((NODE_BLK, D), lambda i: (i, 0)),
            pl.BlockSpec((NODE_BLK, NC, HH), lambda i: (i, 0, 0)),
            pl.BlockSpec((NODE_BLK, NC, HH), lambda i: (i, 0, 0)),
            pl.BlockSpec((D, H), lambda i: (0, 0)),
            pl.BlockSpec((H,), lambda i: (0,)),
            pl.BlockSpec((D, H), lambda i: (0, 0)),
            pl.BlockSpec((H,), lambda i: (0,)),
            pl.BlockSpec((H, D), lambda i: (0, 0)),
            pl.BlockSpec((D,), lambda i: (0,)),
        ],
        out_specs=pl.BlockSpec((NODE_BLK, D), lambda i: (i, 0)),
        out_shape=jax.ShapeDtypeStruct((N, D), jnp.float32),
    )(node_feat, agg_a, agg_b, Wpd, bpd, Wpe, bpe, Wp, bp)


def kernel(node_feat, edge_index, edge_attr, We1, be1, We2, be2, Ws, bs, Wd, bd,
           Wt1, bt1, Wt2, bt2, Wpd, bpd, Wpe, bpe, Wp, bp):
    src = edge_index[0]
    dst = edge_index[1]
    s, d = _node_precompute(node_feat, Ws, bs, Wd, bd)
    We2b = We2.astype(jnp.bfloat16)
    Wt1b = Wt1.astype(jnp.bfloat16)
    Wt2b = Wt2.astype(jnp.bfloat16)
    zeros_stripe = jnp.zeros((NSTRIPE, HH), jnp.float32)

    ea_t = edge_attr.T
    m_chunks = []
    for kk in range(K):
        gs, gd = _sc_gather(s, d, src, dst, kk)
        m = _edge_mlp(ea_t, gs, gd, We1, be1, We2b, be2, Wt1b, bt1,
                      Wt2b, bt2, kk)
        m_chunks.append(m)

    agg_a = _sc_scatter_add(m_chunks[:SCATTER_SPLIT],
                            list(range(SCATTER_SPLIT)), dst, zeros_stripe)
    agg_b = _sc_scatter_add(m_chunks[SCATTER_SPLIT:],
                            list(range(SCATTER_SPLIT, K)), dst, zeros_stripe)
    return _final_stage(node_feat, agg_a, agg_b, Wpd, bpd, Wpe, bpe, Wp, bp)


# speedup vs baseline: 5.1873x; 1.0327x over previous
import functools

import jax
import jax.numpy as jnp
from jax import lax
from jax.experimental import pallas as pl
from jax.experimental.pallas import tpu as pltpu
from jax.experimental.pallas import tpu_sc as plsc

N = 10000
E = 160000
D = 256
DE = 16
H = 256

K = 5
EC = E // K

NODE_BLK = 1000
EDGE_BLK = 1280

NC = 2
NS = 16
NW = NC * NS
GB = 200
SB = 200
HH = H // 2
NSTRIPE = N // NS

SCATTER_SPLIT = 3


def _pack_bf16_pair(lo, hi):
    ulo = lax.bitcast_convert_type(lo, jnp.uint32)
    uhi = lax.bitcast_convert_type(hi, jnp.uint32)
    rlo = ulo + jnp.uint32(0x7FFF) + ((ulo >> jnp.uint32(16)) & jnp.uint32(1))
    rhi = uhi + jnp.uint32(0x7FFF) + ((uhi >> jnp.uint32(16)) & jnp.uint32(1))
    packed = (rhi & jnp.uint32(0xFFFF0000)) | (rlo >> jnp.uint32(16))
    return lax.bitcast_convert_type(packed, jnp.float32)


def _unpack_bf16_pair(w):
    u = lax.bitcast_convert_type(w, jnp.uint32)
    lo = lax.bitcast_convert_type(u << jnp.uint32(16), jnp.float32)
    hi = lax.bitcast_convert_type(u & jnp.uint32(0xFFFF0000), jnp.float32)
    return lo, hi


def _node_pre_body(nf_ref, ws_ref, bs_ref, wd_ref, bd_ref, s_ref, d_ref):
    nf = nf_ref[...]
    s = jnp.dot(nf, ws_ref[...], preferred_element_type=jnp.float32) + bs_ref[...]
    d = jnp.dot(nf, wd_ref[...], preferred_element_type=jnp.float32) + bd_ref[...]
    s_ref[...] = _pack_bf16_pair(s[:, :HH], s[:, HH:])
    d_ref[...] = _pack_bf16_pair(d[:, :HH], d[:, HH:])


def _node_precompute(node_feat, Ws, bs, Wd, bd):
    return pl.pallas_call(
        _node_pre_body,
        grid=(N // NODE_BLK,),
        in_specs=[
            pl.BlockSpec((NODE_BLK, D), lambda i: (i, 0)),
            pl.BlockSpec((D, H), lambda i: (0, 0)),
            pl.BlockSpec((H,), lambda i: (0,)),
            pl.BlockSpec((D, H), lambda i: (0, 0)),
            pl.BlockSpec((H,), lambda i: (0,)),
        ],
        out_specs=[
            pl.BlockSpec((NODE_BLK, HH), lambda i: (i, 0)),
            pl.BlockSpec((NODE_BLK, HH), lambda i: (i, 0)),
        ],
        out_shape=[
            jax.ShapeDtypeStruct((N, HH), jnp.float32),
            jax.ShapeDtypeStruct((N, HH), jnp.float32),
        ],
    )(node_feat, Ws, bs, Wd, bd)


def _sc_gather(s, d, src_r, dst_r, kk):
    per_w = EC // NW
    n_it = per_w // GB
    rows_per_chunk = EC // GB

    mesh = plsc.VectorSubcoreMesh(core_axis_name="c", subcore_axis_name="s")

    @functools.partial(
        pl.kernel,
        mesh=mesh,
        out_type=[
            jax.ShapeDtypeStruct((EC, HH), jnp.float32),
            jax.ShapeDtypeStruct((EC, HH), jnp.float32),
        ],
        scratch_types=[
            pltpu.VMEM((per_w,), jnp.int32),
            pltpu.VMEM((per_w,), jnp.int32),
            pltpu.VMEM((GB, HH), jnp.float32),
            pltpu.VMEM((GB, HH), jnp.float32),
            pltpu.VMEM((GB, HH), jnp.float32),
            pltpu.VMEM((GB, HH), jnp.float32),
        ]
        + [pltpu.SemaphoreType.DMA] * 8,
    )
    def k(s_hbm, d_hbm, src_hbm, dst_hbm, gs_hbm, gd_hbm,
          idx_s, idx_d, rs0, rs1, rd0, rd1,
          gs0, gs1, gd0, gd1, ws0, ws1, wd0, wd1):
        wid = lax.axis_index("s") * NC + lax.axis_index("c")
        base = wid * per_w

        pltpu.sync_copy(src_hbm.at[pl.ds(kk * EC + base, per_w)], idx_s)
        pltpu.sync_copy(dst_hbm.at[pl.ds(kk * EC + base, per_w)], idx_d)

        rows_s = (rs0, rs1)
        rows_d = (rd0, rd1)
        gsem_s = (gs0, gs1)
        gsem_d = (gd0, gd1)
        wsem_s = (ws0, ws1)
        wsem_d = (wd0, wd1)
        gh_s = [None] * n_it
        gh_d = [None] * n_it
        wh_s = [None] * n_it
        wh_d = [None] * n_it

        def start_gather(i):
            b = i & 1
            gh_s[i] = pltpu.async_copy(
                s_hbm.at[idx_s.at[pl.ds(i * GB, GB)]], rows_s[b], gsem_s[b])
            gh_d[i] = pltpu.async_copy(
                d_hbm.at[idx_d.at[pl.ds(i * GB, GB)]], rows_d[b], gsem_d[b])

        start_gather(0)
        for i in range(n_it):
            b = i & 1
            if i + 1 < n_it:
                if i >= 1:
                    wh_s[i - 1].wait()
                    wh_d[i - 1].wait()
                start_gather(i + 1)
            gh_s[i].wait()
            gh_d[i].wait()
            off = base + i * GB
            wh_s[i] = pltpu.async_copy(rows_s[b], gs_hbm.at[pl.ds(off, GB)], wsem_s[b])
            wh_d[i] = pltpu.async_copy(rows_d[b], gd_hbm.at[pl.ds(off, GB)], wsem_d[b])
        for i in range(max(0, n_it - 2), n_it):
            wh_s[i].wait()
            wh_d[i].wait()

    return k(s, d, src_r, dst_r)


def _edge_mlp_body(ea_ref, gs_ref, gd_ref, we1_ref, be1_ref, we2_ref, be2_ref,
                   wt1_ref, bt1_ref, wt2_ref, bt2_ref, m_ref):
    h1 = jnp.maximum(
        lax.dot_general(ea_ref[...], we1_ref[...], (((0,), (0,)), ((), ())),
                        preferred_element_type=jnp.float32) + be1_ref[...], 0.0)
    ec = jnp.dot(h1.astype(jnp.bfloat16), we2_ref[...],
                 preferred_element_type=jnp.float32) + be2_ref[...]
    gs_lo, gs_hi = _unpack_bf16_pair(gs_ref[...])
    gd_lo, gd_hi = _unpack_bf16_pair(gd_ref[...])
    m1_lo = jnp.maximum(ec[:, :HH] + gs_lo + gd_lo, 0.0)
    m1_hi = jnp.maximum(ec[:, HH:] + gs_hi + gd_hi, 0.0)
    m1 = jnp.concatenate([m1_lo, m1_hi], axis=1)
    m2 = jnp.maximum(jnp.dot(m1.astype(jnp.bfloat16), wt1_ref[...],
                             preferred_element_type=jnp.float32) + bt1_ref[...], 0.0)
    m_ref[...] = jnp.dot(m2.astype(jnp.bfloat16), wt2_ref[...],
                         preferred_element_type=jnp.float32) + bt2_ref[...]


def _edge_mlp(edge_attr, gs, gd, We1, be1, We2b, be2, Wt1b, bt1, Wt2b, bt2, kk):
    nb = EC // EDGE_BLK
    return pl.pallas_call(
        _edge_mlp_body,
        grid=(nb,),
        in_specs=[
            pl.BlockSpec((DE, EDGE_BLK), lambda i, kk=kk: (0, i + kk * nb)),
            pl.BlockSpec((EDGE_BLK, HH), lambda i: (i, 0)),
            pl.BlockSpec((EDGE_BLK, HH), lambda i: (i, 0)),
            pl.BlockSpec((DE, H), lambda i: (0, 0)),
            pl.BlockSpec((H,), lambda i: (0,)),
            pl.BlockSpec((H, H), lambda i: (0, 0)),
            pl.BlockSpec((H,), lambda i: (0,)),
            pl.BlockSpec((H, H), lambda i: (0, 0)),
            pl.BlockSpec((H,), lambda i: (0,)),
            pl.BlockSpec((H, D), lambda i: (0, 0)),
            pl.BlockSpec((D,), lambda i: (0,)),
        ],
        out_specs=pl.BlockSpec((EDGE_BLK, D), lambda i: (i, 0)),
        out_shape=jax.ShapeDtypeStruct((EC, D), jnp.float32),
    )(edge_attr, gs, gd, We1, be1, We2b, be2, Wt1b, bt1, Wt2b, bt2)


def _sc_scatter_add(m_chunks, chunk_ids, dst_r, zeros_stripe):
    per_t = EC // NS
    n_it = per_t // SB
    rows_per_chunk = EC // SB
    nchunks = len(m_chunks)

    mesh = plsc.VectorSubcoreMesh(core_axis_name="c", subcore_axis_name="s")

    @functools.partial(
        pl.kernel,
        mesh=mesh,
        out_type=jax.ShapeDtypeStruct((N, NC, HH), jnp.float32),
        scratch_types=[
            pltpu.VMEM((SB,), jnp.int32),
            pltpu.VMEM((SB, HH), jnp.float32),
            pltpu.VMEM_SHARED((N, HH), jnp.float32),
            pltpu.SemaphoreType.DMA,
            pltpu.SemaphoreType.DMA,
        ],
    )
    def k(*refs):
        m_hbms = refs[:nchunks]
        (dst_hbm, z_hbm, out_hbm, idx_v, blk_v, acc, sem_i, sem_m) = refs[nchunks:]
        c = lax.axis_index("c")
        t = lax.axis_index("s")
        col = c * HH

        pltpu.sync_copy(z_hbm, acc.at[pl.ds(t * NSTRIPE, NSTRIPE)])
        plsc.subcore_barrier()

        for j, m_hbm in enumerate(m_hbms):
            kk = chunk_ids[j]

            def body(i, carry, m_hbm=m_hbm, kk=kk):
                off = t * per_t + i * SB
                cp_i = pltpu.async_copy(
                    dst_hbm.at[pl.ds(kk * EC + off, SB)], idx_v, sem_i)
                cp_m = pltpu.async_copy(
                    m_hbm.at[pl.ds(off, SB), pl.ds(col, HH)], blk_v, sem_m)
                cp_i.wait()
                cp_m.wait()
                pltpu.sync_copy(blk_v, acc.at[idx_v], add=True)
                return carry

            lax.fori_loop(0, n_it, body, 0)

        plsc.subcore_barrier()

        pltpu.sync_copy(acc.at[pl.ds(t * NSTRIPE, NSTRIPE)],
                        out_hbm.at[pl.ds(t * NSTRIPE, NSTRIPE), c])

    return k(*m_chunks, dst_r, zeros_stripe)


def _final_body(nf_ref, agg_a_ref, agg_b_ref, wpd_ref, bpd_ref, wpe_ref,
                bpe_ref, wp_ref, bp_ref, out_ref):
    a = agg_a_ref[...] + agg_b_ref[...]
    agg = jnp.concatenate([a[:, 0, :], a[:, 1, :]], axis=1)
    z = (jnp.dot(nf_ref[...], wpd_ref[...], preferred_element_type=jnp.float32)
         + bpd_ref[...]
         + jnp.dot(agg, wpe_ref[...], preferred_element_type=jnp.float32)
         + bpe_ref[...])
    out_ref[...] = jnp.dot(jnp.maximum(z, 0.0), wp_ref[...],
                           preferred_element_type=jnp.float32) + bp_ref[...]


def _final_stage(node_feat, agg_a, agg_b, Wpd, bpd, Wpe, bpe, Wp, bp):
    return pl.pallas_call(
        _final_body,
        grid=(N // NODE_BLK,),
        in_specs=[
            pl.BlockSpec((NODE_BLK, D), lambda i: (i, 0)),
            pl.BlockSpec((NODE_BLK, NC, HH), lambda i: (i, 0, 0)),
            pl.BlockSpec((NODE_BLK, NC, HH), lambda i: (i, 0, 0)),
            pl.BlockSpec((D, H), lambda i: (0, 0)),
            pl.BlockSpec((H,), lambda i: (0,)),
            pl.BlockSpec((D, H), lambda i: (0, 0)),
            pl.BlockSpec((H,), lambda i: (0,)),
            pl.BlockSpec((H, D), lambda i: (0, 0)),
            pl.BlockSpec((D,), lambda i: (0,)),
        ],
        out_specs=pl.BlockSpec((NODE_BLK, D), lambda i: (i, 0)),
        out_shape=jax.ShapeDtypeStruct((N, D), jnp.float32),
    )(node_feat, agg_a, agg_b, Wpd, bpd, Wpe, bpe, Wp, bp)


def kernel(node_feat, edge_index, edge_attr, We1, be1, We2, be2, Ws, bs, Wd, bd,
           Wt1, bt1, Wt2, bt2, Wpd, bpd, Wpe, bpe, Wp, bp):
    src = edge_index[0]
    dst = edge_index[1]
    s, d = _node_precompute(node_feat, Ws, bs, Wd, bd)
    We2b = We2.astype(jnp.bfloat16)
    Wt1b = Wt1.astype(jnp.bfloat16)
    Wt2b = Wt2.astype(jnp.bfloat16)
    zeros_stripe = jnp.zeros((NSTRIPE, HH), jnp.float32)

    ea_t = edge_attr.T
    m_chunks = []
    for kk in range(K):
        gs, gd = _sc_gather(s, d, src, dst, kk)
        m = _edge_mlp(ea_t, gs, gd, We1, be1, We2b, be2, Wt1b, bt1,
                      Wt2b, bt2, kk)
        m_chunks.append(m)

    agg_a = _sc_scatter_add(m_chunks[:SCATTER_SPLIT],
                            list(range(SCATTER_SPLIT)), dst, zeros_stripe)
    agg_b = _sc_scatter_add(m_chunks[SCATTER_SPLIT:],
                            list(range(SCATTER_SPLIT, K)), dst, zeros_stripe)
    return _final_stage(node_feat, agg_a, agg_b, Wpd, bpd, Wpe, bpe, Wp, bp)
